# baseline (speedup 1.0000x reference)
import jax
import jax.numpy as jnp
from jax import lax
from jax.experimental import pallas as pl
from jax.experimental.pallas import tpu as pltpu
from jax.experimental.pallas import tpu_sc as plsc

D = 16
C = 1024
B = 16384
NC = 2
NS = 16
L = 16
CH = C // NC
RPT = B // NS
CK = 64
NCK = RPT // CK
G = CK // L


def _body(mu_hbm, sig_hbm, mu_t_hbm, sig_t_hbm, idx_hbm,
          out_mu, out_sig,
          idx_v, buf, accl, cntr, rbuf, ftab, fout,
          part):
    f32 = jnp.float32
    cid = lax.axis_index("c")
    sid = lax.axis_index("s")
    col0 = cid * CH
    row0 = sid * RPT
    lanes = lax.iota(jnp.int32, L)

    def zrow(dd, c):
        for j in range(CH // L):
            accl[dd, pl.ds(j * L, L)] = jnp.zeros((L,), f32)
        return c
    lax.fori_loop(0, 2 * D + 1, zrow, 0)
    cntr[...] = jnp.zeros((L,), f32)

    pltpu.sync_copy(idx_hbm.at[pl.ds(row0, RPT)], idx_v)

    def superchunk(i, c):
        i2 = i // 2
        p = i % 2
        r = row0 + i2 * CK

        @pl.when(p == 0)
        def _():
            pltpu.sync_copy(mu_hbm.at[pl.ds(r, CK), pl.ds(col0, CH)], buf)

        @pl.when(p == 1)
        def _():
            pltpu.sync_copy(sig_hbm.at[pl.ds(r, CK), pl.ds(col0, CH)], buf)

        doff = p * D

        def grp(g, c2):
            drow = idx_v[pl.ds(i2 * CK + g * L, L)]
            cm = jnp.zeros((L,), f32)
            for rr in range(L):
                d0 = jnp.sum(jnp.where(lanes == rr, drow, 0))
                cm = cm + jnp.where(lanes == d0, 1.0, 0.0)
                d = d0 + doff
                for j in range(CH // L):
                    s = pl.ds(j * L, L)
                    plsc.addupdate(accl.at[d, s], buf[g * L + rr, s])
            cntr[...] = cntr[...] + cm
            return c2

        lax.fori_loop(0, G, grp, 0)
        return c

    lax.fori_loop(0, 2 * NCK, superchunk, 0)

    accl[2 * D, pl.ds(0, L)] = cntr[...]
    pltpu.sync_copy(accl, part.at[:, sid])
    plsc.subcore_barrier()

    pltpu.sync_copy(part.at[2 * D], rbuf)
    cntv = rbuf[0, pl.ds(0, L)]
    for t in range(1, NS):
        cntv = cntv + rbuf[t, pl.ds(0, L)]
    cscal = jnp.sum(jnp.where(lanes == sid, cntv, 0.0)) * 0.5
    denomv = jnp.maximum(jnp.full((L,), cscal), 1.0)
    presv = jnp.full((L,), cscal) > 0.0

    pltpu.sync_copy(part.at[sid], rbuf)
    pltpu.sync_copy(mu_t_hbm.at[sid, pl.ds(col0, CH)], ftab)
    for j in range(CH // L):
        s = pl.ds(j * L, L)
        a = rbuf[0, s]
        for t in range(1, NS):
            a = a + rbuf[t, s]
        fout[s] = jnp.where(presv, a / denomv, ftab[s])
    pltpu.sync_copy(fout, out_mu.at[sid, pl.ds(col0, CH)])

    pltpu.sync_copy(part.at[D + sid], rbuf)
    pltpu.sync_copy(sig_t_hbm.at[sid, pl.ds(col0, CH)], ftab)
    for j in range(CH // L):
        s = pl.ds(j * L, L)
        a = rbuf[0, s]
        for t in range(1, NS):
            a = a + rbuf[t, s]
        fout[s] = jnp.where(presv, a / denomv, ftab[s])
    pltpu.sync_copy(fout, out_sig.at[sid, pl.ds(col0, CH)])


@jax.jit
def _run(mu, sig, mu_table, sig_table, domain_idx):
    f32 = jnp.float32
    k = pl.kernel(
        _body,
        out_type=(jax.ShapeDtypeStruct((D, C), f32),
                  jax.ShapeDtypeStruct((D, C), f32)),
        mesh=plsc.VectorSubcoreMesh(core_axis_name="c", subcore_axis_name="s"),
        scratch_types=[
            pltpu.VMEM((RPT,), jnp.int32),
            pltpu.VMEM((CK, CH), f32),
            pltpu.VMEM((2 * D + 1, CH), f32),
            pltpu.VMEM((L,), f32),
            pltpu.VMEM((NS, CH), f32),
            pltpu.VMEM((CH,), f32),
            pltpu.VMEM((CH,), f32),
            pltpu.VMEM_SHARED((2 * D + 1, NS, CH), f32),
        ],
        compiler_params=pltpu.CompilerParams(needs_layout_passes=False),
    )
    return k(mu, sig, mu_table, sig_table, domain_idx)


def kernel(mu, sig, mu_table, sig_table, domain_idx, layer_idx):
    del layer_idx
    return _run(mu, sig, mu_table, sig_table, domain_idx)
